# baseline (speedup 1.0000x reference)
import jax
import jax.numpy as jnp
from jax import lax
from jax.experimental import pallas as pl
from jax.experimental.pallas import tpu as pltpu
from jax.experimental.pallas import tpu_sc as plsc

N = 50000
N_PAD = 50176
E = 800000
EP = 819200
R = EP // 128
NTILES = 16
NCORES = 2
RPT = R // NTILES
ZR = N_PAD // NTILES
CHB = 3
RPT_B = 402
R_B = RPT_B * NTILES
EP_B = R_B * 128
NCHB = RPT_B // CHB
NTB = NCHB // 2
CHD = 5
NCHD = RPT // CHD
NTD = NCHD // 2
UF = 32
IF = 43
H = 64
HH = 32
BE = 8192
BLK = BE // 4
F32 = jnp.float32

_MESH = plsc.VectorSubcoreMesh(core_axis_name="c", subcore_axis_name="s")
_SC_PARAMS = pltpu.CompilerParams(use_tc_tiling_on_sc=False)


def _segsum_body(mu_lo, mu_hi, mi_lo, mi_hi, idx3, zeros,
                 acci_lo, acci_hi, accu_lo, accu_hi,
                 ib0, rows0, ib1, rows1, acc_sh,
                 semg0, semg1, sems0, sems1):
    cid = lax.axis_index("c")
    sid = lax.axis_index("s")
    tables = ((mu_lo, mi_lo), (mu_hi, mi_hi))
    outs = ((acci_lo, accu_lo), (acci_hi, accu_hi))
    sets = ((ib0, rows0, semg0, sems0), (ib1, rows1, semg1, sems1))
    for d in range(2):
        gdim, sdim = (0, 1) if d == 0 else (1, 0)
        pltpu.sync_copy(zeros, acc_sh.at[pl.ds(sid * ZR, ZR)])
        plsc.subcore_barrier()
        for c in range(NCORES):
            @pl.when(cid == c)
            def _():
                tbl = tables[c][d]

                def g_descs(s):
                    ib, rows, semg, _ = s
                    return [pltpu.make_async_copy(
                        tbl.at[ib.at[j, gdim]],
                        rows.at[pl.ds(j * 128, 128)], semg)
                        for j in range(CHB)]

                def s_descs(s):
                    ib, rows, _, sems = s
                    return [pltpu.make_async_copy(
                        rows.at[pl.ds(j * 128, 128)],
                        acc_sh.at[ib.at[j, sdim]], sems)
                        for j in range(CHB)]

                def load_fire(k, s):
                    row0 = sid * RPT_B + k * CHB
                    pltpu.sync_copy(idx3.at[pl.ds(row0, CHB)], s[0])
                    for d_ in g_descs(s):
                        d_.start()

                def finish(s):
                    for d_ in g_descs(s):
                        d_.wait()
                    for d_ in s_descs(s):
                        d_.start(add=True)

                def drain(s):
                    for d_ in s_descs(s):
                        d_.wait()

                load_fire(0, sets[0])
                load_fire(1, sets[1])

                def pair(t, carry):
                    k0 = 2 * t
                    finish(sets[0])
                    finish(sets[1])

                    @pl.when(t < NTB - 1)
                    def _():
                        drain(sets[0])
                        load_fire(k0 + 2, sets[0])
                        drain(sets[1])
                        load_fire(k0 + 3, sets[1])
                    return carry

                lax.fori_loop(0, NTB, pair, 0)
                drain(sets[0])
                drain(sets[1])
        plsc.subcore_barrier()
        for c in range(NCORES):
            @pl.when(cid == c)
            def _():
                pltpu.sync_copy(acc_sh.at[pl.ds(sid * ZR, ZR)],
                                outs[c][d].at[pl.ds(sid * ZR, ZR)])
        plsc.subcore_barrier()


_SEG = pl.kernel(
    _segsum_body,
    out_type=[jax.ShapeDtypeStruct((N_PAD, HH), F32)] * 4,
    mesh=_MESH,
    compiler_params=_SC_PARAMS,
    scratch_types=[
        pltpu.VMEM((CHB, 2, 128), jnp.int32),
        pltpu.VMEM((CHB * 128, HH), F32),
        pltpu.VMEM((CHB, 2, 128), jnp.int32),
        pltpu.VMEM((CHB * 128, HH), F32),
        pltpu.VMEM_SHARED((N_PAD, HH), F32),
        pltpu.SemaphoreType.DMA,
        pltpu.SemaphoreType.DMA,
        pltpu.SemaphoreType.DMA,
        pltpu.SemaphoreType.DMA,
    ],
)


def _edge_gather_body(au_lo, au_hi, ai_lo, ai_hi, idx3,
                      gu_lo, gu_hi, gi_lo, gi_hi,
                      ib0, ru0, ri0, ib1, ru1, ri1,
                      semg0, semg1, semw0, semw1):
    cid = lax.axis_index("c")
    sid = lax.axis_index("s")
    cb = sid // 4
    lane0 = cb * HH
    sets = ((ib0, ru0, ri0, semg0, semw0),
            (ib1, ru1, ri1, semg1, semw1))
    for c in range(NCORES):
        @pl.when(cid == c)
        def _():
            au = (au_lo, au_hi)[c]
            ai = (ai_lo, ai_hi)[c]
            gu = (gu_lo, gu_hi)[c]
            gi = (gi_lo, gi_hi)[c]

            def gather_descs(k, s):
                ib, ru, ri, semg, _ = s
                descs = []
                for j in range(CHD):
                    descs.append(pltpu.make_async_copy(
                        au.at[ib.at[j, 0]], ru.at[pl.ds(j * 128, 128)], semg))
                    descs.append(pltpu.make_async_copy(
                        ai.at[ib.at[j, 1]], ri.at[pl.ds(j * 128, 128)], semg))
                return descs

            def load_fire(k, s):
                ib = s[0]
                row0 = sid * RPT + k * CHD
                pltpu.sync_copy(idx3.at[pl.ds(row0, CHD)], ib)
                for d_ in gather_descs(k, s):
                    d_.start()

            def write_descs(k, s):
                _, ru, ri, _, semw = s
                p0 = (sid % 4) * RPT * 128 + k * CHD * 128
                return [
                    pltpu.make_async_copy(
                        ru, gu.at[pl.ds(p0, CHD * 128), pl.ds(lane0, HH)],
                        semw),
                    pltpu.make_async_copy(
                        ri, gi.at[pl.ds(p0, CHD * 128), pl.ds(lane0, HH)],
                        semw),
                ]

            def finish_chunk(k, s):
                for d_ in gather_descs(k, s):
                    d_.wait()
                for d_ in write_descs(k, s):
                    d_.start()

            def drain_writes(k, s):
                for d_ in write_descs(k, s):
                    d_.wait()

            load_fire(0, sets[0])
            load_fire(1, sets[1])

            def pair(t, carry):
                k0 = 2 * t
                finish_chunk(k0, sets[0])
                finish_chunk(k0 + 1, sets[1])

                @pl.when(t < NTD - 1)
                def _():
                    drain_writes(k0, sets[0])
                    load_fire(k0 + 2, sets[0])
                    drain_writes(k0 + 1, sets[1])
                    load_fire(k0 + 3, sets[1])
                return carry

            lax.fori_loop(0, NTD, pair, 0)
            drain_writes(NCHD - 2, sets[0])
            drain_writes(NCHD - 1, sets[1])


_EDGE = pl.kernel(
    _edge_gather_body,
    out_type=[jax.ShapeDtypeStruct((EP // 4, 128), F32)] * 4,
    mesh=_MESH,
    compiler_params=_SC_PARAMS,
    scratch_types=[
        pltpu.VMEM((CHD, 2, 128), jnp.int32),
        pltpu.VMEM((CHD * 128, HH), F32),
        pltpu.VMEM((CHD * 128, HH), F32),
        pltpu.VMEM((CHD, 2, 128), jnp.int32),
        pltpu.VMEM((CHD * 128, HH), F32),
        pltpu.VMEM((CHD * 128, HH), F32),
        pltpu.SemaphoreType.DMA,
        pltpu.SemaphoreType.DMA,
        pltpu.SemaphoreType.DMA,
        pltpu.SemaphoreType.DMA,
    ],
)


def _msg_body(xu, xi, wb, wbd, mu_lo, mu_hi, mi_lo, mi_hi):
    mu = jnp.dot(xu[...], wb[...], preferred_element_type=F32)
    mi = jnp.dot(xi[...], wbd[...], preferred_element_type=F32)
    mu_lo[...] = mu[:, :HH]
    mu_hi[...] = mu[:, HH:]
    mi_lo[...] = mi[:, :HH]
    mi_hi[...] = mi[:, HH:]


def _stage_msg(xu, xi, wb, wbd):
    nb = N_PAD // ZR
    return pl.pallas_call(
        _msg_body,
        grid=(nb,),
        in_specs=[
            pl.BlockSpec((ZR, UF), lambda i: (i, 0)),
            pl.BlockSpec((ZR, IF), lambda i: (i, 0)),
            pl.BlockSpec((UF, H), lambda i: (0, 0)),
            pl.BlockSpec((IF, H), lambda i: (0, 0)),
        ],
        out_specs=[pl.BlockSpec((ZR, HH), lambda i: (i, 0))] * 4,
        out_shape=[jax.ShapeDtypeStruct((N_PAD, HH), F32)] * 4,
    )(xu, xi, wb, wbd)


def _a_body(x, lo, hi, w1x, w1h, a_lo, a_hi):
    acc = jnp.concatenate([lo[...], hi[...]], axis=1)
    a = (jnp.dot(x[...], w1x[...], preferred_element_type=F32)
         + jnp.dot(acc, w1h[...], preferred_element_type=F32))
    a_lo[...] = a[:, :HH]
    a_hi[...] = a[:, HH:]


def _stage_a(x, lo, hi, w1x, w1h, nf):
    nb = N_PAD // ZR
    return pl.pallas_call(
        _a_body,
        grid=(nb,),
        in_specs=[
            pl.BlockSpec((ZR, nf), lambda i: (i, 0)),
            pl.BlockSpec((ZR, HH), lambda i: (i, 0)),
            pl.BlockSpec((ZR, HH), lambda i: (i, 0)),
            pl.BlockSpec((nf, H), lambda i: (0, 0)),
            pl.BlockSpec((H, H), lambda i: (0, 0)),
        ],
        out_specs=[pl.BlockSpec((ZR, HH), lambda i: (i, 0))] * 2,
        out_shape=[jax.ShapeDtypeStruct((N_PAD, HH), F32)] * 2,
    )(x, lo, hi, w1x, w1h)


def _mlp_body(gul, guh, gil, gih, d_lo, d_hi, w3r, b2r, b3r, b1r,
              bbuyr, bbuyedr, w1uh, w1ih, out):
    c1 = (b1r[...]
          + jnp.dot(bbuyedr[...], w1uh[...], preferred_element_type=F32)
          + jnp.dot(bbuyr[...], w1ih[...], preferred_element_type=F32))
    c1_lo = jnp.concatenate([c1[:, :HH]] * 4, axis=1)
    c1_hi = jnp.concatenate([c1[:, HH:]] * 4, axis=1)
    b2p = jnp.concatenate([b2r[...]] * 4, axis=1)
    z1_lo = jnp.maximum(gul[...] + gil[...] + c1_lo, 0.0)
    z1_hi = jnp.maximum(guh[...] + gih[...] + c1_hi, 0.0)
    z2 = jnp.maximum(
        jnp.dot(z1_lo.astype(jnp.bfloat16), d_lo[...],
                preferred_element_type=F32)
        + jnp.dot(z1_hi.astype(jnp.bfloat16), d_hi[...],
                  preferred_element_type=F32) + b2p, 0.0)
    lp = jnp.dot(z2, w3r[...], preferred_element_type=F32)
    out[...] = jax.nn.sigmoid(jnp.transpose(lp) + b3r[0, 0])


def _stage_mlp(gul, guh, gil, gih, d_lo, d_hi, w3r, b2r, b3r, b1r,
               bbuyr, bbuyedr, w1uh, w1ih):
    nb = (EP // 4) // BLK
    return pl.pallas_call(
        _mlp_body,
        grid=(nb,),
        in_specs=[
            pl.BlockSpec((BLK, 128), lambda i: (i, 0)),
            pl.BlockSpec((BLK, 128), lambda i: (i, 0)),
            pl.BlockSpec((BLK, 128), lambda i: (i, 0)),
            pl.BlockSpec((BLK, 128), lambda i: (i, 0)),
            pl.BlockSpec((128, 256), lambda i: (0, 0)),
            pl.BlockSpec((128, 256), lambda i: (0, 0)),
            pl.BlockSpec((256, 4), lambda i: (0, 0)),
            pl.BlockSpec((1, H), lambda i: (0, 0)),
            pl.BlockSpec((1, 1), lambda i: (0, 0)),
            pl.BlockSpec((1, H), lambda i: (0, 0)),
            pl.BlockSpec((1, H), lambda i: (0, 0)),
            pl.BlockSpec((1, H), lambda i: (0, 0)),
            pl.BlockSpec((H, H), lambda i: (0, 0)),
            pl.BlockSpec((H, H), lambda i: (0, 0)),
        ],
        out_specs=pl.BlockSpec((4, BLK), lambda i: (0, i)),
        out_shape=jax.ShapeDtypeStruct((4, EP // 4), F32),
    )(gul, guh, gil, gih, d_lo, d_hi, w3r, b2r, b3r, b1r,
      bbuyr, bbuyedr, w1uh, w1ih)


def kernel(user_x, item_x, edge_index, labels,
           W_buy, b_buy, W_buyed, b_buyed,
           W1, b1, W2, b2, W3, b3):
    src = edge_index[0].astype(jnp.int32)
    dst = edge_index[1].astype(jnp.int32)
    padb = jnp.full((EP_B - E,), N, jnp.int32)
    idx3_b = jnp.stack([jnp.concatenate([src, padb]).reshape(R_B, 128),
                        jnp.concatenate([dst, padb]).reshape(R_B, 128)],
                       axis=1)
    padd = jnp.full((EP - E,), N, jnp.int32)
    idx3_d = jnp.stack([jnp.concatenate([src, padd]).reshape(R, 128),
                        jnp.concatenate([dst, padd]).reshape(R, 128)],
                       axis=1)
    xu = jnp.pad(user_x, ((0, N_PAD - N), (0, 0)))
    xi = jnp.pad(item_x, ((0, N_PAD - N), (0, 0)))
    zeros = jnp.zeros((ZR, HH), F32)

    mu_lo, mu_hi, mi_lo, mi_hi = _stage_msg(xu, xi, W_buy, W_buyed)
    w1u = W1[:UF]
    w1i = W1[UF:UF + IF]
    w1uh = W1[UF + IF:UF + IF + H]
    w1ih = W1[UF + IF + H:]
    acci_lo, acci_hi, accu_lo, accu_hi = _SEG(
        mu_lo, mu_hi, mi_lo, mi_hi, idx3_b, zeros)
    ai_lo, ai_hi = _stage_a(xi, acci_lo, acci_hi, w1i, w1ih, IF)
    au_lo, au_hi = _stage_a(xu, accu_lo, accu_hi, w1u, w1uh, UF)
    gu_lo, gu_hi, gi_lo, gi_hi = _EDGE(au_lo, au_hi, ai_lo, ai_hi, idx3_d)
    eye4 = jnp.eye(4, dtype=F32)
    d_lo = jnp.kron(eye4, W2[:HH]).astype(jnp.bfloat16)
    d_hi = jnp.kron(eye4, W2[HH:]).astype(jnp.bfloat16)
    w3blk = jnp.kron(eye4, W3)
    scores_p = _stage_mlp(
        gu_lo, gu_hi, gi_lo, gi_hi, d_lo, d_hi, w3blk,
        b2.reshape(1, H), b3.reshape(1, 1), b1.reshape(1, H),
        b_buy.reshape(1, H), b_buyed.reshape(1, H), w1uh, w1ih)
    return scores_p.reshape(EP)[:E], labels


# speedup vs baseline: 1.0366x; 1.0366x over previous
import jax
import jax.numpy as jnp
from jax import lax
from jax.experimental import pallas as pl
from jax.experimental.pallas import tpu as pltpu
from jax.experimental.pallas import tpu_sc as plsc

N = 50000
N_PAD = 50176
E = 800000
EP = 819200
R = EP // 128
NTILES = 16
NCORES = 2
RPT = R // NTILES
ZR = N_PAD // NTILES
CHB = 3
RPT_B = 402
R_B = RPT_B * NTILES
EP_B = R_B * 128
NCHB = RPT_B // CHB
NTB = NCHB // 2
CHD = 4
NCHD = RPT // CHD
NTD = NCHD // 2
UF = 32
IF = 43
H = 64
HH = 32
BE = 8192
BLK = BE // 4
F32 = jnp.float32

_MESH = plsc.VectorSubcoreMesh(core_axis_name="c", subcore_axis_name="s")
_SC_PARAMS = pltpu.CompilerParams(use_tc_tiling_on_sc=False)


def _segsum_body(mu_lo, mu_hi, mi_lo, mi_hi, idx3, zeros,
                 acci_lo, acci_hi, accu_lo, accu_hi,
                 ib0, rows0, ib1, rows1, acc_sh,
                 semg0, semg1, sems0, sems1):
    cid = lax.axis_index("c")
    sid = lax.axis_index("s")
    tables = ((mu_lo, mi_lo), (mu_hi, mi_hi))
    outs = ((acci_lo, accu_lo), (acci_hi, accu_hi))
    sets = ((ib0, rows0, semg0, sems0), (ib1, rows1, semg1, sems1))
    for d in range(2):
        gdim, sdim = (0, 1) if d == 0 else (1, 0)
        pltpu.sync_copy(zeros, acc_sh.at[pl.ds(sid * ZR, ZR)])
        plsc.subcore_barrier()
        for c in range(NCORES):
            @pl.when(cid == c)
            def _():
                tbl = tables[c][d]

                def g_descs(s):
                    ib, rows, semg, _ = s
                    return [pltpu.make_async_copy(
                        tbl.at[ib.at[j, gdim]],
                        rows.at[pl.ds(j * 128, 128)], semg)
                        for j in range(CHB)]

                def s_descs(s):
                    ib, rows, _, sems = s
                    return [pltpu.make_async_copy(
                        rows.at[pl.ds(j * 128, 128)],
                        acc_sh.at[ib.at[j, sdim]], sems)
                        for j in range(CHB)]

                def load_fire(k, s):
                    row0 = sid * RPT_B + k * CHB
                    pltpu.sync_copy(idx3.at[pl.ds(row0, CHB)], s[0])
                    for d_ in g_descs(s):
                        d_.start()

                def finish(s):
                    for d_ in g_descs(s):
                        d_.wait()
                    for d_ in s_descs(s):
                        d_.start(add=True)

                def drain(s):
                    for d_ in s_descs(s):
                        d_.wait()

                load_fire(0, sets[0])
                load_fire(1, sets[1])

                def pair(t, carry):
                    k0 = 2 * t
                    finish(sets[0])
                    finish(sets[1])

                    @pl.when(t < NTB - 1)
                    def _():
                        drain(sets[0])
                        load_fire(k0 + 2, sets[0])
                        drain(sets[1])
                        load_fire(k0 + 3, sets[1])
                    return carry

                lax.fori_loop(0, NTB, pair, 0)
                drain(sets[0])
                drain(sets[1])
        plsc.subcore_barrier()
        for c in range(NCORES):
            @pl.when(cid == c)
            def _():
                pltpu.sync_copy(acc_sh.at[pl.ds(sid * ZR, ZR)],
                                outs[c][d].at[pl.ds(sid * ZR, ZR)])
        plsc.subcore_barrier()


_SEG = pl.kernel(
    _segsum_body,
    out_type=[jax.ShapeDtypeStruct((N_PAD, HH), F32)] * 4,
    mesh=_MESH,
    compiler_params=_SC_PARAMS,
    scratch_types=[
        pltpu.VMEM((CHB, 2, 128), jnp.int32),
        pltpu.VMEM((CHB * 128, HH), F32),
        pltpu.VMEM((CHB, 2, 128), jnp.int32),
        pltpu.VMEM((CHB * 128, HH), F32),
        pltpu.VMEM_SHARED((N_PAD, HH), F32),
        pltpu.SemaphoreType.DMA,
        pltpu.SemaphoreType.DMA,
        pltpu.SemaphoreType.DMA,
        pltpu.SemaphoreType.DMA,
    ],
)


def _edge_gather_body(au_lo, au_hi, ai_lo, ai_hi, idx3,
                      gu_lo, gu_hi, gi_lo, gi_hi,
                      ib0, ru0, ri0, ib1, ru1, ri1,
                      semg0, semg1, semw0, semw1):
    cid = lax.axis_index("c")
    sid = lax.axis_index("s")
    cb = sid // 4
    lane0 = cb * HH
    sets = ((ib0, ru0, ri0, semg0, semw0),
            (ib1, ru1, ri1, semg1, semw1))
    for c in range(NCORES):
        @pl.when(cid == c)
        def _():
            au = (au_lo, au_hi)[c]
            ai = (ai_lo, ai_hi)[c]
            gu = (gu_lo, gu_hi)[c]
            gi = (gi_lo, gi_hi)[c]

            def gather_descs(k, s):
                ib, ru, ri, semg, _ = s
                descs = []
                for j in range(CHD):
                    descs.append(pltpu.make_async_copy(
                        au.at[ib.at[j, 0]], ru.at[pl.ds(j * 128, 128)], semg))
                    descs.append(pltpu.make_async_copy(
                        ai.at[ib.at[j, 1]], ri.at[pl.ds(j * 128, 128)], semg))
                return descs

            def load_fire(k, s):
                ib = s[0]
                row0 = sid * RPT + k * CHD
                pltpu.sync_copy(idx3.at[pl.ds(row0, CHD)], ib)
                for d_ in gather_descs(k, s):
                    d_.start()

            def write_descs(k, s):
                _, ru, ri, _, semw = s
                p0 = (sid % 4) * RPT * 128 + k * CHD * 128
                return [
                    pltpu.make_async_copy(
                        ru, gu.at[pl.ds(p0, CHD * 128), pl.ds(lane0, HH)],
                        semw),
                    pltpu.make_async_copy(
                        ri, gi.at[pl.ds(p0, CHD * 128), pl.ds(lane0, HH)],
                        semw),
                ]

            def finish_chunk(k, s):
                for d_ in gather_descs(k, s):
                    d_.wait()
                for d_ in write_descs(k, s):
                    d_.start()

            def drain_writes(k, s):
                for d_ in write_descs(k, s):
                    d_.wait()

            load_fire(0, sets[0])
            load_fire(1, sets[1])

            def pair(t, carry):
                k0 = 2 * t
                finish_chunk(k0, sets[0])
                finish_chunk(k0 + 1, sets[1])

                @pl.when(t < NTD - 1)
                def _():
                    drain_writes(k0, sets[0])
                    load_fire(k0 + 2, sets[0])
                    drain_writes(k0 + 1, sets[1])
                    load_fire(k0 + 3, sets[1])
                return carry

            lax.fori_loop(0, NTD, pair, 0)
            drain_writes(NCHD - 2, sets[0])
            drain_writes(NCHD - 1, sets[1])


_EDGE = pl.kernel(
    _edge_gather_body,
    out_type=[jax.ShapeDtypeStruct((EP // 4, 128), F32)] * 4,
    mesh=_MESH,
    compiler_params=_SC_PARAMS,
    scratch_types=[
        pltpu.VMEM((CHD, 2, 128), jnp.int32),
        pltpu.VMEM((CHD * 128, HH), F32),
        pltpu.VMEM((CHD * 128, HH), F32),
        pltpu.VMEM((CHD, 2, 128), jnp.int32),
        pltpu.VMEM((CHD * 128, HH), F32),
        pltpu.VMEM((CHD * 128, HH), F32),
        pltpu.SemaphoreType.DMA,
        pltpu.SemaphoreType.DMA,
        pltpu.SemaphoreType.DMA,
        pltpu.SemaphoreType.DMA,
    ],
)


def _msg_body(xu, xi, wb, wbd, mu_lo, mu_hi, mi_lo, mi_hi):
    mu = jnp.dot(xu[...], wb[...], preferred_element_type=F32)
    mi = jnp.dot(xi[...], wbd[...], preferred_element_type=F32)
    mu_lo[...] = mu[:, :HH]
    mu_hi[...] = mu[:, HH:]
    mi_lo[...] = mi[:, :HH]
    mi_hi[...] = mi[:, HH:]


def _stage_msg(xu, xi, wb, wbd):
    nb = N_PAD // ZR
    return pl.pallas_call(
        _msg_body,
        grid=(nb,),
        in_specs=[
            pl.BlockSpec((ZR, UF), lambda i: (i, 0)),
            pl.BlockSpec((ZR, IF), lambda i: (i, 0)),
            pl.BlockSpec((UF, H), lambda i: (0, 0)),
            pl.BlockSpec((IF, H), lambda i: (0, 0)),
        ],
        out_specs=[pl.BlockSpec((ZR, HH), lambda i: (i, 0))] * 4,
        out_shape=[jax.ShapeDtypeStruct((N_PAD, HH), F32)] * 4,
    )(xu, xi, wb, wbd)


def _a_body(xu, ul, uh, xi, il, ih, w1u, w1uh, w1i, w1ih,
            au_lo, au_hi, ai_lo, ai_hi):
    accu = jnp.concatenate([ul[...], uh[...]], axis=1)
    acci = jnp.concatenate([il[...], ih[...]], axis=1)
    au = (jnp.dot(xu[...], w1u[...], preferred_element_type=F32)
          + jnp.dot(accu, w1uh[...], preferred_element_type=F32))
    ai = (jnp.dot(xi[...], w1i[...], preferred_element_type=F32)
          + jnp.dot(acci, w1ih[...], preferred_element_type=F32))
    au_lo[...] = au[:, :HH]
    au_hi[...] = au[:, HH:]
    ai_lo[...] = ai[:, :HH]
    ai_hi[...] = ai[:, HH:]


def _stage_a(xu, ul, uh, xi, il, ih, w1u, w1uh, w1i, w1ih):
    nb = N_PAD // ZR
    return pl.pallas_call(
        _a_body,
        grid=(nb,),
        in_specs=[
            pl.BlockSpec((ZR, UF), lambda i: (i, 0)),
            pl.BlockSpec((ZR, HH), lambda i: (i, 0)),
            pl.BlockSpec((ZR, HH), lambda i: (i, 0)),
            pl.BlockSpec((ZR, IF), lambda i: (i, 0)),
            pl.BlockSpec((ZR, HH), lambda i: (i, 0)),
            pl.BlockSpec((ZR, HH), lambda i: (i, 0)),
            pl.BlockSpec((UF, H), lambda i: (0, 0)),
            pl.BlockSpec((H, H), lambda i: (0, 0)),
            pl.BlockSpec((IF, H), lambda i: (0, 0)),
            pl.BlockSpec((H, H), lambda i: (0, 0)),
        ],
        out_specs=[pl.BlockSpec((ZR, HH), lambda i: (i, 0))] * 4,
        out_shape=[jax.ShapeDtypeStruct((N_PAD, HH), F32)] * 4,
    )(xu, ul, uh, xi, il, ih, w1u, w1uh, w1i, w1ih)


def _mlp_body(gul, guh, gil, gih, d_lo, d_hi, w3r, b2r, b3r, b1r,
              bbuyr, bbuyedr, w1uh, w1ih, out):
    c1 = (b1r[...]
          + jnp.dot(bbuyedr[...], w1uh[...], preferred_element_type=F32)
          + jnp.dot(bbuyr[...], w1ih[...], preferred_element_type=F32))
    c1_lo = jnp.concatenate([c1[:, :HH]] * 4, axis=1)
    c1_hi = jnp.concatenate([c1[:, HH:]] * 4, axis=1)
    b2p = jnp.concatenate([b2r[...]] * 4, axis=1)
    z1_lo = jnp.maximum(gul[...] + gil[...] + c1_lo, 0.0)
    z1_hi = jnp.maximum(guh[...] + gih[...] + c1_hi, 0.0)
    z2 = jnp.maximum(
        jnp.dot(z1_lo.astype(jnp.bfloat16), d_lo[...],
                preferred_element_type=F32)
        + jnp.dot(z1_hi.astype(jnp.bfloat16), d_hi[...],
                  preferred_element_type=F32) + b2p, 0.0)
    lp = jnp.dot(z2, w3r[...], preferred_element_type=F32)
    out[...] = jax.nn.sigmoid(jnp.transpose(lp) + b3r[0, 0])


def _stage_mlp(gul, guh, gil, gih, d_lo, d_hi, w3r, b2r, b3r, b1r,
               bbuyr, bbuyedr, w1uh, w1ih):
    nb = (EP // 4) // BLK
    return pl.pallas_call(
        _mlp_body,
        grid=(nb,),
        in_specs=[
            pl.BlockSpec((BLK, 128), lambda i: (i, 0)),
            pl.BlockSpec((BLK, 128), lambda i: (i, 0)),
            pl.BlockSpec((BLK, 128), lambda i: (i, 0)),
            pl.BlockSpec((BLK, 128), lambda i: (i, 0)),
            pl.BlockSpec((128, 256), lambda i: (0, 0)),
            pl.BlockSpec((128, 256), lambda i: (0, 0)),
            pl.BlockSpec((256, 4), lambda i: (0, 0)),
            pl.BlockSpec((1, H), lambda i: (0, 0)),
            pl.BlockSpec((1, 1), lambda i: (0, 0)),
            pl.BlockSpec((1, H), lambda i: (0, 0)),
            pl.BlockSpec((1, H), lambda i: (0, 0)),
            pl.BlockSpec((1, H), lambda i: (0, 0)),
            pl.BlockSpec((H, H), lambda i: (0, 0)),
            pl.BlockSpec((H, H), lambda i: (0, 0)),
        ],
        out_specs=pl.BlockSpec((4, BLK), lambda i: (0, i)),
        out_shape=jax.ShapeDtypeStruct((4, EP // 4), F32),
    )(gul, guh, gil, gih, d_lo, d_hi, w3r, b2r, b3r, b1r,
      bbuyr, bbuyedr, w1uh, w1ih)


def kernel(user_x, item_x, edge_index, labels,
           W_buy, b_buy, W_buyed, b_buyed,
           W1, b1, W2, b2, W3, b3):
    src = edge_index[0].astype(jnp.int32)
    dst = edge_index[1].astype(jnp.int32)
    padb = jnp.full((EP_B - E,), N, jnp.int32)
    idx3_b = jnp.stack([jnp.concatenate([src, padb]).reshape(R_B, 128),
                        jnp.concatenate([dst, padb]).reshape(R_B, 128)],
                       axis=1)
    padd = jnp.full((EP - E,), N, jnp.int32)
    idx3_d = jnp.stack([jnp.concatenate([src, padd]).reshape(R, 128),
                        jnp.concatenate([dst, padd]).reshape(R, 128)],
                       axis=1)
    xu = jnp.pad(user_x, ((0, N_PAD - N), (0, 0)))
    xi = jnp.pad(item_x, ((0, N_PAD - N), (0, 0)))
    zeros = jnp.zeros((ZR, HH), F32)

    mu_lo, mu_hi, mi_lo, mi_hi = _stage_msg(xu, xi, W_buy, W_buyed)
    w1u = W1[:UF]
    w1i = W1[UF:UF + IF]
    w1uh = W1[UF + IF:UF + IF + H]
    w1ih = W1[UF + IF + H:]
    acci_lo, acci_hi, accu_lo, accu_hi = _SEG(
        mu_lo, mu_hi, mi_lo, mi_hi, idx3_b, zeros)
    au_lo, au_hi, ai_lo, ai_hi = _stage_a(
        xu, accu_lo, accu_hi, xi, acci_lo, acci_hi, w1u, w1uh, w1i, w1ih)
    gu_lo, gu_hi, gi_lo, gi_hi = _EDGE(au_lo, au_hi, ai_lo, ai_hi, idx3_d)
    eye4 = jnp.eye(4, dtype=F32)
    d_lo = jnp.kron(eye4, W2[:HH]).astype(jnp.bfloat16)
    d_hi = jnp.kron(eye4, W2[HH:]).astype(jnp.bfloat16)
    w3blk = jnp.kron(eye4, W3)
    scores_p = _stage_mlp(
        gu_lo, gu_hi, gi_lo, gi_hi, d_lo, d_hi, w3blk,
        b2.reshape(1, H), b3.reshape(1, 1), b1.reshape(1, H),
        b_buy.reshape(1, H), b_buyed.reshape(1, H), w1uh, w1ih)
    return scores_p.reshape(EP)[:E], labels


# speedup vs baseline: 1.0369x; 1.0003x over previous
import jax
import jax.numpy as jnp
from jax import lax
from jax.experimental import pallas as pl
from jax.experimental.pallas import tpu as pltpu
from jax.experimental.pallas import tpu_sc as plsc

N = 50000
N_PAD = 50176
E = 800000
EP = 819200
R = EP // 128
NTILES = 16
NCORES = 2
RPT = R // NTILES
ZR = N_PAD // NTILES
CHB = 3
RPT_B = 402
R_B = RPT_B * NTILES
EP_B = R_B * 128
NCHB = RPT_B // CHB
NTB = NCHB // 2
CHD = 4
NCHD = RPT // CHD
NTD = NCHD // 2
UF = 32
IF = 43
H = 64
HH = 32
BE = 8192
BLK = BE // 4
F32 = jnp.float32

_MESH = plsc.VectorSubcoreMesh(core_axis_name="c", subcore_axis_name="s",
                               num_cores=NCORES, num_subcores=NTILES)
_SC_PARAMS = pltpu.CompilerParams(use_tc_tiling_on_sc=False)


def _segsum_body(mu_lo, mu_hi, mi_lo, mi_hi, idx3, zeros,
                 acci_lo, acci_hi, accu_lo, accu_hi,
                 ib0, rows0, ib1, rows1, acc_sh,
                 semg0, semg1, sems0, sems1):
    cid = lax.axis_index("c")
    sid = lax.axis_index("s")
    tables = ((mu_lo, mi_lo), (mu_hi, mi_hi))
    outs = ((acci_lo, accu_lo), (acci_hi, accu_hi))
    sets = ((ib0, rows0, semg0, sems0), (ib1, rows1, semg1, sems1))
    for d in range(2):
        gdim, sdim = (0, 1) if d == 0 else (1, 0)
        pltpu.sync_copy(zeros, acc_sh.at[pl.ds(sid * ZR, ZR)])
        plsc.subcore_barrier()
        for c in range(NCORES):
            @pl.when(cid == c)
            def _():
                tbl = tables[c][d]

                def g_descs(s):
                    ib, rows, semg, _ = s
                    return [pltpu.make_async_copy(
                        tbl.at[ib.at[j, gdim]],
                        rows.at[pl.ds(j * 128, 128)], semg)
                        for j in range(CHB)]

                def s_descs(s):
                    ib, rows, _, sems = s
                    return [pltpu.make_async_copy(
                        rows.at[pl.ds(j * 128, 128)],
                        acc_sh.at[ib.at[j, sdim]], sems)
                        for j in range(CHB)]

                def load_fire(k, s):
                    row0 = sid * RPT_B + k * CHB
                    pltpu.sync_copy(idx3.at[pl.ds(row0, CHB)], s[0])
                    for d_ in g_descs(s):
                        d_.start()

                def finish(s):
                    for d_ in g_descs(s):
                        d_.wait()
                    for d_ in s_descs(s):
                        d_.start(add=True)

                def drain(s):
                    for d_ in s_descs(s):
                        d_.wait()

                load_fire(0, sets[0])
                load_fire(1, sets[1])

                def pair(t, carry):
                    k0 = 2 * t
                    finish(sets[0])
                    finish(sets[1])

                    @pl.when(t < NTB - 1)
                    def _():
                        drain(sets[0])
                        load_fire(k0 + 2, sets[0])
                        drain(sets[1])
                        load_fire(k0 + 3, sets[1])
                    return carry

                lax.fori_loop(0, NTB, pair, 0)
                drain(sets[0])
                drain(sets[1])
        plsc.subcore_barrier()
        for c in range(NCORES):
            @pl.when(cid == c)
            def _():
                pltpu.sync_copy(acc_sh.at[pl.ds(sid * ZR, ZR)],
                                outs[c][d].at[pl.ds(sid * ZR, ZR)])
        plsc.subcore_barrier()


_SEG = pl.kernel(
    _segsum_body,
    out_type=[jax.ShapeDtypeStruct((N_PAD, HH), F32)] * 4,
    mesh=_MESH,
    compiler_params=_SC_PARAMS,
    scratch_types=[
        pltpu.VMEM((CHB, 2, 128), jnp.int32),
        pltpu.VMEM((CHB * 128, HH), F32),
        pltpu.VMEM((CHB, 2, 128), jnp.int32),
        pltpu.VMEM((CHB * 128, HH), F32),
        pltpu.VMEM_SHARED((N_PAD, HH), F32),
        pltpu.SemaphoreType.DMA,
        pltpu.SemaphoreType.DMA,
        pltpu.SemaphoreType.DMA,
        pltpu.SemaphoreType.DMA,
    ],
)


def _edge_gather_body(au_lo, au_hi, ai_lo, ai_hi, idx3,
                      gu_lo, gu_hi, gi_lo, gi_hi,
                      ib0, ru0, ri0, ib1, ru1, ri1,
                      semg0, semg1, semw0, semw1):
    cid = lax.axis_index("c")
    sid = lax.axis_index("s")
    cb = sid // 4
    lane0 = cb * HH
    sets = ((ib0, ru0, ri0, semg0, semw0),
            (ib1, ru1, ri1, semg1, semw1))
    for c in range(NCORES):
        @pl.when(cid == c)
        def _():
            au = (au_lo, au_hi)[c]
            ai = (ai_lo, ai_hi)[c]
            gu = (gu_lo, gu_hi)[c]
            gi = (gi_lo, gi_hi)[c]

            def gather_descs(k, s):
                ib, ru, ri, semg, _ = s
                descs = []
                for j in range(CHD):
                    descs.append(pltpu.make_async_copy(
                        au.at[ib.at[j, 0]], ru.at[pl.ds(j * 128, 128)], semg))
                    descs.append(pltpu.make_async_copy(
                        ai.at[ib.at[j, 1]], ri.at[pl.ds(j * 128, 128)], semg))
                return descs

            def load_fire(k, s):
                ib = s[0]
                row0 = sid * RPT + k * CHD
                pltpu.sync_copy(idx3.at[pl.ds(row0, CHD)], ib)
                for d_ in gather_descs(k, s):
                    d_.start()

            def write_descs(k, s):
                _, ru, ri, _, semw = s
                p0 = (sid % 4) * RPT * 128 + k * CHD * 128
                return [
                    pltpu.make_async_copy(
                        ru, gu.at[pl.ds(p0, CHD * 128), pl.ds(lane0, HH)],
                        semw),
                    pltpu.make_async_copy(
                        ri, gi.at[pl.ds(p0, CHD * 128), pl.ds(lane0, HH)],
                        semw),
                ]

            def finish_chunk(k, s):
                for d_ in gather_descs(k, s):
                    d_.wait()
                for d_ in write_descs(k, s):
                    d_.start()

            def drain_writes(k, s):
                for d_ in write_descs(k, s):
                    d_.wait()

            load_fire(0, sets[0])
            load_fire(1, sets[1])

            def pair(t, carry):
                k0 = 2 * t
                finish_chunk(k0, sets[0])
                finish_chunk(k0 + 1, sets[1])

                @pl.when(t < NTD - 1)
                def _():
                    drain_writes(k0, sets[0])
                    load_fire(k0 + 2, sets[0])
                    drain_writes(k0 + 1, sets[1])
                    load_fire(k0 + 3, sets[1])
                return carry

            lax.fori_loop(0, NTD, pair, 0)
            drain_writes(NCHD - 2, sets[0])
            drain_writes(NCHD - 1, sets[1])


_EDGE = pl.kernel(
    _edge_gather_body,
    out_type=[jax.ShapeDtypeStruct((EP // 4, 128), F32)] * 4,
    mesh=_MESH,
    compiler_params=_SC_PARAMS,
    scratch_types=[
        pltpu.VMEM((CHD, 2, 128), jnp.int32),
        pltpu.VMEM((CHD * 128, HH), F32),
        pltpu.VMEM((CHD * 128, HH), F32),
        pltpu.VMEM((CHD, 2, 128), jnp.int32),
        pltpu.VMEM((CHD * 128, HH), F32),
        pltpu.VMEM((CHD * 128, HH), F32),
        pltpu.SemaphoreType.DMA,
        pltpu.SemaphoreType.DMA,
        pltpu.SemaphoreType.DMA,
        pltpu.SemaphoreType.DMA,
    ],
)


def _msg_body(xu, xi, wb, wbd, mu_lo, mu_hi, mi_lo, mi_hi):
    mu = jnp.dot(xu[...], wb[...], preferred_element_type=F32)
    mi = jnp.dot(xi[...], wbd[...], preferred_element_type=F32)
    mu_lo[...] = mu[:, :HH]
    mu_hi[...] = mu[:, HH:]
    mi_lo[...] = mi[:, :HH]
    mi_hi[...] = mi[:, HH:]


def _stage_msg(xu, xi, wb, wbd):
    nb = N_PAD // ZR
    return pl.pallas_call(
        _msg_body,
        grid=(nb,),
        in_specs=[
            pl.BlockSpec((ZR, UF), lambda i: (i, 0)),
            pl.BlockSpec((ZR, IF), lambda i: (i, 0)),
            pl.BlockSpec((UF, H), lambda i: (0, 0)),
            pl.BlockSpec((IF, H), lambda i: (0, 0)),
        ],
        out_specs=[pl.BlockSpec((ZR, HH), lambda i: (i, 0))] * 4,
        out_shape=[jax.ShapeDtypeStruct((N_PAD, HH), F32)] * 4,
    )(xu, xi, wb, wbd)


def _a_body(xu, ul, uh, xi, il, ih, w1u, w1uh, w1i, w1ih,
            au_lo, au_hi, ai_lo, ai_hi):
    accu = jnp.concatenate([ul[...], uh[...]], axis=1)
    acci = jnp.concatenate([il[...], ih[...]], axis=1)
    au = (jnp.dot(xu[...], w1u[...], preferred_element_type=F32)
          + jnp.dot(accu, w1uh[...], preferred_element_type=F32))
    ai = (jnp.dot(xi[...], w1i[...], preferred_element_type=F32)
          + jnp.dot(acci, w1ih[...], preferred_element_type=F32))
    au_lo[...] = au[:, :HH]
    au_hi[...] = au[:, HH:]
    ai_lo[...] = ai[:, :HH]
    ai_hi[...] = ai[:, HH:]


def _stage_a(xu, ul, uh, xi, il, ih, w1u, w1uh, w1i, w1ih):
    nb = N_PAD // ZR
    return pl.pallas_call(
        _a_body,
        grid=(nb,),
        in_specs=[
            pl.BlockSpec((ZR, UF), lambda i: (i, 0)),
            pl.BlockSpec((ZR, HH), lambda i: (i, 0)),
            pl.BlockSpec((ZR, HH), lambda i: (i, 0)),
            pl.BlockSpec((ZR, IF), lambda i: (i, 0)),
            pl.BlockSpec((ZR, HH), lambda i: (i, 0)),
            pl.BlockSpec((ZR, HH), lambda i: (i, 0)),
            pl.BlockSpec((UF, H), lambda i: (0, 0)),
            pl.BlockSpec((H, H), lambda i: (0, 0)),
            pl.BlockSpec((IF, H), lambda i: (0, 0)),
            pl.BlockSpec((H, H), lambda i: (0, 0)),
        ],
        out_specs=[pl.BlockSpec((ZR, HH), lambda i: (i, 0))] * 4,
        out_shape=[jax.ShapeDtypeStruct((N_PAD, HH), F32)] * 4,
    )(xu, ul, uh, xi, il, ih, w1u, w1uh, w1i, w1ih)


def _mlp_body(gul, guh, gil, gih, d_lo, d_hi, w3r, b2r, b3r, b1r,
              bbuyr, bbuyedr, w1uh, w1ih, out):
    c1 = (b1r[...]
          + jnp.dot(bbuyedr[...], w1uh[...], preferred_element_type=F32)
          + jnp.dot(bbuyr[...], w1ih[...], preferred_element_type=F32))
    c1_lo = jnp.concatenate([c1[:, :HH]] * 4, axis=1)
    c1_hi = jnp.concatenate([c1[:, HH:]] * 4, axis=1)
    b2p = jnp.concatenate([b2r[...]] * 4, axis=1)
    z1_lo = jnp.maximum(gul[...] + gil[...] + c1_lo, 0.0)
    z1_hi = jnp.maximum(guh[...] + gih[...] + c1_hi, 0.0)
    z2 = jnp.maximum(
        jnp.dot(z1_lo.astype(jnp.bfloat16), d_lo[...],
                preferred_element_type=F32)
        + jnp.dot(z1_hi.astype(jnp.bfloat16), d_hi[...],
                  preferred_element_type=F32) + b2p, 0.0)
    lp = jnp.dot(z2, w3r[...], preferred_element_type=F32)
    out[...] = jax.nn.sigmoid(jnp.transpose(lp) + b3r[0, 0])


def _stage_mlp(gul, guh, gil, gih, d_lo, d_hi, w3r, b2r, b3r, b1r,
               bbuyr, bbuyedr, w1uh, w1ih):
    nb = (EP // 4) // BLK
    return pl.pallas_call(
        _mlp_body,
        grid=(nb,),
        in_specs=[
            pl.BlockSpec((BLK, 128), lambda i: (i, 0)),
            pl.BlockSpec((BLK, 128), lambda i: (i, 0)),
            pl.BlockSpec((BLK, 128), lambda i: (i, 0)),
            pl.BlockSpec((BLK, 128), lambda i: (i, 0)),
            pl.BlockSpec((128, 256), lambda i: (0, 0)),
            pl.BlockSpec((128, 256), lambda i: (0, 0)),
            pl.BlockSpec((256, 4), lambda i: (0, 0)),
            pl.BlockSpec((1, H), lambda i: (0, 0)),
            pl.BlockSpec((1, 1), lambda i: (0, 0)),
            pl.BlockSpec((1, H), lambda i: (0, 0)),
            pl.BlockSpec((1, H), lambda i: (0, 0)),
            pl.BlockSpec((1, H), lambda i: (0, 0)),
            pl.BlockSpec((H, H), lambda i: (0, 0)),
            pl.BlockSpec((H, H), lambda i: (0, 0)),
        ],
        out_specs=pl.BlockSpec((4, BLK), lambda i: (0, i)),
        out_shape=jax.ShapeDtypeStruct((4, EP // 4), F32),
    )(gul, guh, gil, gih, d_lo, d_hi, w3r, b2r, b3r, b1r,
      bbuyr, bbuyedr, w1uh, w1ih)


def kernel(user_x, item_x, edge_index, labels,
           W_buy, b_buy, W_buyed, b_buyed,
           W1, b1, W2, b2, W3, b3):
    src = edge_index[0].astype(jnp.int32)
    dst = edge_index[1].astype(jnp.int32)
    padb = jnp.full((EP_B - E,), N, jnp.int32)
    idx3_b = jnp.stack([jnp.concatenate([src, padb]).reshape(R_B, 128),
                        jnp.concatenate([dst, padb]).reshape(R_B, 128)],
                       axis=1)
    padd = jnp.full((EP - E,), N, jnp.int32)
    idx3_d = jnp.stack([jnp.concatenate([src, padd]).reshape(R, 128),
                        jnp.concatenate([dst, padd]).reshape(R, 128)],
                       axis=1)
    xu = jnp.pad(user_x, ((0, N_PAD - N), (0, 0)))
    xi = jnp.pad(item_x, ((0, N_PAD - N), (0, 0)))
    zeros = jnp.zeros((ZR, HH), F32)

    mu_lo, mu_hi, mi_lo, mi_hi = _stage_msg(xu, xi, W_buy, W_buyed)
    w1u = W1[:UF]
    w1i = W1[UF:UF + IF]
    w1uh = W1[UF + IF:UF + IF + H]
    w1ih = W1[UF + IF + H:]
    acci_lo, acci_hi, accu_lo, accu_hi = _SEG(
        mu_lo, mu_hi, mi_lo, mi_hi, idx3_b, zeros)
    au_lo, au_hi, ai_lo, ai_hi = _stage_a(
        xu, accu_lo, accu_hi, xi, acci_lo, acci_hi, w1u, w1uh, w1i, w1ih)
    gu_lo, gu_hi, gi_lo, gi_hi = _EDGE(au_lo, au_hi, ai_lo, ai_hi, idx3_d)
    eye4 = jnp.eye(4, dtype=F32)
    d_lo = jnp.kron(eye4, W2[:HH]).astype(jnp.bfloat16)
    d_hi = jnp.kron(eye4, W2[HH:]).astype(jnp.bfloat16)
    w3blk = jnp.kron(eye4, W3)
    scores_p = _stage_mlp(
        gu_lo, gu_hi, gi_lo, gi_hi, d_lo, d_hi, w3blk,
        b2.reshape(1, H), b3.reshape(1, 1), b1.reshape(1, H),
        b_buy.reshape(1, H), b_buyed.reshape(1, H), w1uh, w1ih)
    return scores_p.reshape(EP)[:E], labels
